# baseline (speedup 1.0000x reference)
import functools

import jax
import jax.numpy as jnp
from jax import lax
from jax.experimental import pallas as pl
from jax.experimental.pallas import tpu as pltpu
from jax.experimental.pallas import tpu_sc as plsc

_VOCAB = 1000000
_DIM = 64
_BATCH = 4096
_HIST = 200

_NC, _NS = 2, 16
_NW = _NC * _NS
_ROWS_PER_W = _BATCH // _NW
_NBUF = 4

_mesh = plsc.VectorSubcoreMesh(
    core_axis_name="c", subcore_axis_name="s",
    num_cores=_NC, num_subcores=_NS,
)


@functools.partial(
    pl.kernel,
    out_type=jax.ShapeDtypeStruct((_BATCH, _HIST, _DIM), jnp.float32),
    mesh=_mesh,
    scratch_types=[
        pltpu.VMEM((_NBUF, _HIST), jnp.int32),
        [pltpu.VMEM((_HIST, _DIM), jnp.float32) for _ in range(_NBUF)],
        [pltpu.SemaphoreType.DMA for _ in range(_NBUF)],
        [pltpu.SemaphoreType.DMA for _ in range(_NBUF)],
    ],
    compiler_params=pltpu.CompilerParams(use_tc_tiling_on_sc=False),
)
def _gather_kernel(idx_hbm, table_hbm, out_hbm, idx_v, rows, gsem, wsem):
    wid = lax.axis_index("s") * _NC + lax.axis_index("c")
    row0 = wid * _ROWS_PER_W

    pltpu.sync_copy(idx_hbm.at[pl.ds(row0, _NBUF)], idx_v)
    for b in range(_NBUF):
        pltpu.async_copy(table_hbm.at[idx_v.at[b]], rows[b], gsem[b])

    @pl.loop(0, _ROWS_PER_W, step=_NBUF)
    def _slab(g0):
        for b in range(_NBUF):
            pltpu.make_async_copy(
                table_hbm.at[idx_v.at[b]], rows[b], gsem[b]).wait()
            pltpu.async_copy(rows[b], out_hbm.at[row0 + g0 + b], wsem[b])

        @pl.when(g0 + _NBUF < _ROWS_PER_W)
        def _next():
            pltpu.sync_copy(
                idx_hbm.at[pl.ds(row0 + g0 + _NBUF, _NBUF)], idx_v)
            for b in range(_NBUF):
                pltpu.make_async_copy(
                    rows[b], out_hbm.at[row0 + g0 + b], wsem[b]).wait()
                pltpu.async_copy(table_hbm.at[idx_v.at[b]], rows[b], gsem[b])

    last0 = row0 + _ROWS_PER_W - _NBUF
    for b in range(_NBUF):
        pltpu.make_async_copy(
            rows[b], out_hbm.at[last0 + b], wsem[b]).wait()


def kernel(inputs, table):
    return _gather_kernel(inputs, table)


# speedup vs baseline: 1.2036x; 1.2036x over previous
import functools

import jax
import jax.numpy as jnp
from jax import lax
from jax.experimental import pallas as pl
from jax.experimental.pallas import tpu as pltpu
from jax.experimental.pallas import tpu_sc as plsc

_VOCAB = 1000000
_DIM = 64
_B = 4096 * 200
_NC, _NS = 2, 16
_NW = 32
_B_PER_W = _B // _NW
_CHUNK = 128
_N_CHUNKS = _B_PER_W // _CHUNK
_NBUF = 4

_mesh = plsc.VectorSubcoreMesh(
    core_axis_name="c", subcore_axis_name="s",
    num_cores=_NC, num_subcores=_NS,
)


@functools.partial(
    pl.kernel,
    out_type=jax.ShapeDtypeStruct((_B, _DIM), jnp.float32),
    mesh=_mesh,
    scratch_types=[
        pltpu.VMEM((2 * _NBUF, _CHUNK), jnp.int32),
        [pltpu.VMEM((_CHUNK, 128), jnp.float32) for _ in range(_NBUF)],
        [pltpu.VMEM((_CHUNK, _DIM), jnp.float32) for _ in range(2)],
        [pltpu.SemaphoreType.DMA for _ in range(_NBUF)],
        [pltpu.SemaphoreType.DMA for _ in range(2)],
    ],
    compiler_params=pltpu.CompilerParams(use_tc_tiling_on_sc=True),
)
def _gather_kernel(idx_hbm, tpad_hbm, out_hbm, idx_v, rows, rows64,
                   gsem, wsem):
    wid = lax.axis_index("s") * _NC + lax.axis_index("c")
    chunk0 = wid * _N_CHUNKS
    base = wid * _B_PER_W

    pltpu.sync_copy(idx_hbm.at[pl.ds(chunk0, _NBUF)],
                    idx_v.at[pl.ds(0, _NBUF)])
    for b in range(_NBUF):
        pltpu.async_copy(tpad_hbm.at[idx_v.at[b]], rows[b], gsem[b])

    @pl.loop(0, _N_CHUNKS, step=_NBUF)
    def _slab(g0):
        for b in range(_NBUF):
            c = b % 2
            pltpu.make_async_copy(
                tpad_hbm.at[idx_v.at[b]], rows[b], gsem[b]).wait()

            @pl.when(g0 + b >= 2)
            def _reuse():
                pltpu.make_async_copy(
                    rows64[c], out_hbm.at[pl.ds(0, _CHUNK)],
                    wsem[c]).wait()

            @pl.loop(0, _CHUNK)
            def _row(i):
                for k in range(_DIM // 16):
                    rows64[c][i, pl.ds(16 * k, 16)] = (
                        rows[b][i, pl.ds(16 * k, 16)])

            pltpu.async_copy(
                rows64[c],
                out_hbm.at[pl.ds(base + (g0 + b) * _CHUNK, _CHUNK)],
                wsem[c])

        @pl.when(g0 + _NBUF < _N_CHUNKS)
        def _next():
            pltpu.sync_copy(
                idx_hbm.at[pl.ds(chunk0 + g0 + _NBUF, _NBUF)],
                idx_v.at[pl.ds(0, _NBUF)])
            for b in range(_NBUF):
                pltpu.async_copy(tpad_hbm.at[idx_v.at[b]], rows[b], gsem[b])

    for c in range(2):
        pltpu.make_async_copy(
            rows64[c], out_hbm.at[pl.ds(0, _CHUNK)], wsem[c]).wait()


def kernel(inputs, table):
    tpad = jnp.pad(table, ((0, 0), (0, 64)))
    idx = inputs.reshape(_B // _CHUNK, _CHUNK)
    out = _gather_kernel(idx, tpad)
    return out.reshape(4096, 200, 64)
